# baseline (speedup 1.0000x reference)
import functools

import jax
import jax.numpy as jnp
from jax.experimental import pallas as pl
from jax.experimental.pallas import tpu as pltpu

_INNER = 2
_MAX_SEG = 32


def _pick_seg(n):
    for d in range(_MAX_SEG, 0, -1):
        if n % d == 0:
            return d
    return n


def _fused_kernel(x_ref, w1_ref, b1_ref, w2_ref, b2_ref,
                  w1t_ref, b1t_ref, w2t_ref, b2t_ref, out_ref,
                  s_acc_ref, m_acc_ref, *, inner, inv_thw, inv_chw):
    f32 = jnp.float32
    j = pl.program_id(1)
    hw = x_ref.shape[1]
    seg = _pick_seg(hw)
    s_acc = None
    m_acc = None
    for k in range(hw // seg):
        blk = x_ref[0, k * seg:(k + 1) * seg]
        ps = jnp.sum(blk, axis=0)
        pm = jnp.max(blk, axis=0)
        s_acc = ps if s_acc is None else s_acc + ps
        m_acc = pm if m_acc is None else jnp.maximum(m_acc, pm)

    @pl.when(j == 0)
    def _():
        s_acc_ref[...] = s_acc
        m_acc_ref[...] = m_acc

    @pl.when(j > 0)
    def _():
        s_acc_ref[...] += s_acc
        m_acc_ref[...] = jnp.maximum(m_acc_ref[...], m_acc)

    @pl.when(j == inner - 1)
    def _():
        S = s_acc_ref[...]
        M = m_acc_ref[...]

        c11 = (((1,), (1,)), ((), ()))
        a1 = jnp.sum(S, axis=0, keepdims=True) * inv_thw
        m1 = jnp.max(M, axis=0, keepdims=True)
        w1 = w1_ref[...]
        b1 = b1_ref[...]
        ha = jnp.maximum(
            jax.lax.dot_general(a1, w1, c11, preferred_element_type=f32)
            + b1, 0.0)
        hm = jnp.maximum(
            jax.lax.dot_general(m1, w1, c11, preferred_element_type=f32)
            + b1, 0.0)
        w2 = w2_ref[...]
        o1 = (jnp.dot(ha, w2, preferred_element_type=f32)
              + jnp.dot(hm, w2, preferred_element_type=f32)
              + 2.0 * b2_ref[...])
        scale = jax.nn.sigmoid(o1)

        pa = jnp.sum(S * scale, axis=1, keepdims=True).T * inv_chw
        pm2 = jnp.max(M * scale, axis=1, keepdims=True).T
        w1t = w1t_ref[...]
        b1t = b1t_ref[...]
        h2a = jnp.maximum(
            jax.lax.dot_general(pa, w1t, c11, preferred_element_type=f32)
            + b1t, 0.0)
        h2m = jnp.maximum(
            jax.lax.dot_general(pm2, w1t, c11, preferred_element_type=f32)
            + b1t, 0.0)
        w2t = w2t_ref[...]
        o2 = (jnp.dot(h2a, w2t, preferred_element_type=f32)
              + jnp.dot(h2m, w2t, preferred_element_type=f32)
              + 2.0 * b2t_ref[...])
        out_ref[...] = jax.nn.sigmoid(o2)[None]


def kernel(x, w1, b1, w2, b2, w1t, b1t, w2t, b2t):
    B, C, T, H, W = x.shape
    HW = H * W
    inner = _INNER if HW % _INNER == 0 else 1
    hw_blk = HW // inner

    xt = jnp.transpose(x, (0, 3, 4, 2, 1)).reshape(B, HW, T, C)

    fused = functools.partial(
        _fused_kernel, inner=inner,
        inv_thw=1.0 / float(T * H * W), inv_chw=1.0 / float(C * H * W))
    zmap = lambda shape: (lambda b, j: tuple(0 for _ in shape))
    full = lambda a: pl.BlockSpec(a.shape, zmap(a.shape))
    w1v, b1t_r, w2t_r, b2t_r = w1.T, b1t.T, w2t.T, b2t.T
    mc2 = pl.pallas_call(
        fused,
        out_shape=jax.ShapeDtypeStruct((B, 1, T), jnp.float32),
        grid=(B, inner),
        in_specs=[pl.BlockSpec((1, hw_blk, T, C), lambda b, j: (b, j, 0, 0)),
                  full(w1v), full(b1), full(w2), full(b2),
                  full(w1t), full(b1t_r), full(w2t_r), full(b2t_r)],
        out_specs=pl.BlockSpec((1, 1, T), lambda b, j: (b, 0, 0)),
        scratch_shapes=[pltpu.VMEM((T, C), jnp.float32),
                        pltpu.VMEM((T, C), jnp.float32)],
        compiler_params=pltpu.CompilerParams(
            dimension_semantics=("arbitrary", "arbitrary")),
    )(xt, w1v, b1, w2, b2, w1t, b1t_r, w2t_r, b2t_r)
    return mc2.reshape(B, T)


# speedup vs baseline: 1.0337x; 1.0337x over previous
import functools

import jax
import jax.numpy as jnp
from jax.experimental import pallas as pl
from jax.experimental.pallas import tpu as pltpu

_SEG = 16


def _fused_kernel(x_ref, w1_ref, b1_ref, w2_ref, b2_ref,
                  w1t_ref, b1t_ref, w2t_ref, b2t_ref, out_ref, *,
                  inv_thw, inv_chw):
    f32 = jnp.float32
    hw = x_ref.shape[1]
    seg = _SEG if hw % _SEG == 0 else hw
    s_acc = None
    m_acc = None
    for k in range(hw // seg):
        blk = x_ref[0, k * seg:(k + 1) * seg]
        ps = jnp.sum(blk, axis=0)
        pm = jnp.max(blk, axis=0)
        s_acc = ps if s_acc is None else s_acc + ps
        m_acc = pm if m_acc is None else jnp.maximum(m_acc, pm)
    S, M = s_acc, m_acc

    c11 = (((1,), (1,)), ((), ()))
    a1 = jnp.sum(S, axis=0, keepdims=True) * inv_thw
    m1 = jnp.max(M, axis=0, keepdims=True)
    w1 = w1_ref[...]
    b1 = b1_ref[...]
    ha = jnp.maximum(
        jax.lax.dot_general(a1, w1, c11, preferred_element_type=f32) + b1, 0.0)
    hm = jnp.maximum(
        jax.lax.dot_general(m1, w1, c11, preferred_element_type=f32) + b1, 0.0)
    w2 = w2_ref[...]
    o1 = (jnp.dot(ha, w2, preferred_element_type=f32)
          + jnp.dot(hm, w2, preferred_element_type=f32)
          + 2.0 * b2_ref[...])
    scale = jax.nn.sigmoid(o1)

    pa = jnp.sum(S * scale, axis=1, keepdims=True).T * inv_chw
    pm2 = jnp.max(M * scale, axis=1, keepdims=True).T
    w1t = w1t_ref[...]
    b1t = b1t_ref[...]
    h2a = jnp.maximum(
        jax.lax.dot_general(pa, w1t, c11, preferred_element_type=f32)
        + b1t, 0.0)
    h2m = jnp.maximum(
        jax.lax.dot_general(pm2, w1t, c11, preferred_element_type=f32)
        + b1t, 0.0)
    w2t = w2t_ref[...]
    o2 = (jnp.dot(h2a, w2t, preferred_element_type=f32)
          + jnp.dot(h2m, w2t, preferred_element_type=f32)
          + 2.0 * b2t_ref[...])
    out_ref[...] = jax.nn.sigmoid(o2)[None]


def kernel(x, w1, b1, w2, b2, w1t, b1t, w2t, b2t):
    B, C, T, H, W = x.shape
    HW = H * W

    xt = jnp.transpose(x, (0, 3, 4, 2, 1)).reshape(B, HW, T, C)

    fused = functools.partial(
        _fused_kernel,
        inv_thw=1.0 / float(T * H * W), inv_chw=1.0 / float(C * H * W))
    zmap = lambda shape: (lambda b: tuple(0 for _ in shape))
    full = lambda a: pl.BlockSpec(a.shape, zmap(a.shape))
    w1v, b1t_r, w2t_r, b2t_r = w1.T, b1t.T, w2t.T, b2t.T
    mc2 = pl.pallas_call(
        fused,
        out_shape=jax.ShapeDtypeStruct((B, 1, T), jnp.float32),
        grid=(B,),
        in_specs=[pl.BlockSpec((1, HW, T, C), lambda b: (b, 0, 0, 0)),
                  full(w1v), full(b1), full(w2), full(b2),
                  full(w1t), full(b1t_r), full(w2t_r), full(b2t_r)],
        out_specs=pl.BlockSpec((1, 1, T), lambda b: (b, 0, 0)),
        compiler_params=pltpu.CompilerParams(
            dimension_semantics=("arbitrary",)),
    )(xt, w1v, b1, w2, b2, w1t, b1t_r, w2t_r, b2t_r)
    return mc2.reshape(B, T)
